# baseline (speedup 1.0000x reference)
import functools

import jax
import jax.numpy as jnp
from jax import lax
from jax.experimental import pallas as pl
from jax.experimental.pallas import tpu as pltpu
from jax.experimental.pallas import tpu_sc as plsc

D = 64
B = 4096 * 200
NC, NS = 2, 16
NW = NC * NS
B_PER_W = B // NW
CHUNK = 512
NCHUNK = B_PER_W // CHUNK


def _emb_body(idx_hbm, table_hbm, out_hbm, idx_v, rows_v, sem):
    wid = lax.axis_index("s") * NC + lax.axis_index("c")
    base = wid * B_PER_W

    def step(g, carry):
        off = base + g * CHUNK
        pltpu.sync_copy(idx_hbm.at[pl.ds(off, CHUNK)], idx_v)
        pltpu.async_copy(table_hbm.at[idx_v], rows_v, sem).wait()
        pltpu.sync_copy(rows_v, out_hbm.at[pl.ds(off, CHUNK)])
        return carry

    lax.fori_loop(0, NCHUNK, step, 0)


@jax.jit
def _embed(idx, table):
    mesh = plsc.VectorSubcoreMesh(core_axis_name="c", subcore_axis_name="s")
    f = functools.partial(
        pl.kernel,
        out_type=jax.ShapeDtypeStruct((B, D), jnp.float32),
        mesh=mesh,
        scratch_types=[
            pltpu.VMEM((CHUNK,), jnp.int32),
            pltpu.VMEM((CHUNK, D), jnp.float32),
            pltpu.SemaphoreType.DMA,
        ],
        compiler_params=pltpu.CompilerParams(use_tc_tiling_on_sc=False),
    )(_emb_body)
    return f(idx, table)


def kernel(text, seq_len, text_embed_weight):
    idx = text.reshape(-1).astype(jnp.int32)
    out = _embed(idx, text_embed_weight)
    return out.reshape(text.shape[0], text.shape[1], D)


# speedup vs baseline: 1.0444x; 1.0444x over previous
import functools

import jax
import jax.numpy as jnp
from jax import lax
from jax.experimental import pallas as pl
from jax.experimental.pallas import tpu as pltpu
from jax.experimental.pallas import tpu_sc as plsc

D = 64
B = 4096 * 200
NC, NS = 2, 16
NW = NC * NS
B_PER_W = B // NW
CHUNK = 512
NCHUNK = B_PER_W // CHUNK
NP = NCHUNK // 2


def _emb_body(idx_hbm, table_hbm, out_hbm, idx_v, rows0, rows1,
              gs0, gs1, ws0, ws1):
    rows = (rows0, rows1)
    gsem = (gs0, gs1)
    wsem = (ws0, ws1)
    wid = lax.axis_index("s") * NC + lax.axis_index("c")
    base = wid * B_PER_W

    pltpu.sync_copy(idx_hbm.at[pl.ds(base, B_PER_W)], idx_v)

    def gather_start(g, b):
        pltpu.async_copy(
            table_hbm.at[idx_v.at[pl.ds(g * CHUNK, CHUNK)]], rows[b], gsem[b])

    def write_start(g, b):
        pltpu.async_copy(
            rows[b], out_hbm.at[pl.ds(base + g * CHUNK, CHUNK)], wsem[b])

    gather_start(0, 0)
    gather_start(1, 1)

    def step(p, carry):
        for b in (0, 1):
            g = 2 * p + b
            pltpu.make_async_copy(
                table_hbm.at[idx_v.at[pl.ds(g * CHUNK, CHUNK)]],
                rows[b], gsem[b]).wait()
            write_start(g, b)
            pltpu.make_async_copy(
                rows[b], out_hbm.at[pl.ds(base + g * CHUNK, CHUNK)],
                wsem[b]).wait()
            gather_start(g + 2, b)
        return carry

    lax.fori_loop(0, NP - 1, step, 0)

    for b in (0, 1):
        g = NCHUNK - 2 + b
        pltpu.make_async_copy(
            table_hbm.at[idx_v.at[pl.ds(g * CHUNK, CHUNK)]],
            rows[b], gsem[b]).wait()
        write_start(g, b)
    for b in (0, 1):
        g = NCHUNK - 2 + b
        pltpu.make_async_copy(
            rows[b], out_hbm.at[pl.ds(base + g * CHUNK, CHUNK)],
            wsem[b]).wait()


@jax.jit
def _embed(idx, table):
    mesh = plsc.VectorSubcoreMesh(core_axis_name="c", subcore_axis_name="s")
    f = functools.partial(
        pl.kernel,
        out_type=jax.ShapeDtypeStruct((B, D), jnp.float32),
        mesh=mesh,
        scratch_types=[
            pltpu.VMEM((B_PER_W,), jnp.int32),
            pltpu.VMEM((CHUNK, D), jnp.float32),
            pltpu.VMEM((CHUNK, D), jnp.float32),
            pltpu.SemaphoreType.DMA,
            pltpu.SemaphoreType.DMA,
            pltpu.SemaphoreType.DMA,
            pltpu.SemaphoreType.DMA,
        ],
        compiler_params=pltpu.CompilerParams(use_tc_tiling_on_sc=False),
    )(_emb_body)
    return f(idx, table)


def kernel(text, seq_len, text_embed_weight):
    idx = text.reshape(-1).astype(jnp.int32)
    out = _embed(idx, text_embed_weight)
    return out.reshape(text.shape[0], text.shape[1], D)
